# baseline (speedup 1.0000x reference)
import dataclasses
import functools

import jax
import jax.numpy as jnp
from jax import lax
from jax.experimental import pallas as pl
from jax.experimental.pallas import tpu as pltpu
from jax.experimental.pallas import tpu_sc as plsc

TOPK = 50
ROWS = 16384
COLS = 1000
BLOCK_ROWS = 256

_NEG_INF = float("-inf")

_SC_CORES = 2
_SC_SUBCORES = 16
_SC_WORKERS = _SC_CORES * _SC_SUBCORES
_SC_LANES = 16
_TAB_PAD = 1024

_N_IDX = ROWS * TOPK
_IDX_PER_W = _N_IDX // _SC_WORKERS


def _topk_body(xn_ref, conf_ref, idx_ref):
    x = xn_ref[...].T
    iota = lax.broadcasted_iota(jnp.int32, x.shape, 0).astype(jnp.float32)
    for j in range(TOPK):
        m = jnp.max(x, axis=0)
        cand = jnp.where(x == m[None, :], iota, float(COLS + 1))
        idx = jnp.min(cand, axis=0)
        hit = cand == idx[None, :]
        conf_ref[j, :] = m
        idx_ref[j, :] = idx.astype(jnp.int32)
        x = jnp.where(hit, _NEG_INF, x)


def _tc_topk(xn):
    grid = (ROWS // BLOCK_ROWS,)
    return pl.pallas_call(
        _topk_body,
        grid=grid,
        in_specs=[pl.BlockSpec((BLOCK_ROWS, COLS), lambda i: (i, 0))],
        out_specs=[
            pl.BlockSpec((TOPK, BLOCK_ROWS), lambda i: (0, i)),
            pl.BlockSpec((TOPK, BLOCK_ROWS), lambda i: (0, i)),
        ],
        out_shape=[
            jax.ShapeDtypeStruct((TOPK, ROWS), jnp.float32),
            jax.ShapeDtypeStruct((TOPK, ROWS), jnp.int32),
        ],
    )(xn)


def _sc_label_gather(table_pad, idx_flat):
    mesh = plsc.VectorSubcoreMesh(core_axis_name="c", subcore_axis_name="s")
    cp = pltpu.CompilerParams()
    if "needs_layout_passes" in pltpu.CompilerParams.__dataclass_fields__:
        cp = dataclasses.replace(cp, needs_layout_passes=False)

    @functools.partial(
        pl.kernel,
        mesh=mesh,
        compiler_params=cp,
        out_type=jax.ShapeDtypeStruct((_N_IDX,), jnp.int32),
        scratch_types=[
            pltpu.VMEM((_TAB_PAD,), jnp.int32),
            pltpu.VMEM((_IDX_PER_W,), jnp.int32),
            pltpu.VMEM((_IDX_PER_W,), jnp.int32),
        ],
    )
    def k(tab_hbm, idx_hbm, out_hbm, tab_v, idx_v, out_v):
        wid = lax.axis_index("s") * _SC_CORES + lax.axis_index("c")
        base = wid * _IDX_PER_W
        pltpu.sync_copy(tab_hbm, tab_v)
        pltpu.sync_copy(idx_hbm.at[pl.ds(base, _IDX_PER_W)], idx_v)

        @pl.loop(0, _IDX_PER_W, step=_SC_LANES)
        def _(i):
            iv = idx_v[pl.ds(i, _SC_LANES)]
            out_v[pl.ds(i, _SC_LANES)] = plsc.load_gather(tab_v, [iv])

        pltpu.sync_copy(out_v, out_hbm.at[pl.ds(base, _IDX_PER_W)])

    return k(table_pad, idx_flat)


@jax.jit
def kernel(x, label_ids):
    conf_t, idx_t = _tc_topk(x)
    conf = conf_t.T
    idx = idx_t.T
    table_pad = jnp.pad(label_ids, (0, _TAB_PAD - COLS))
    labels = _sc_label_gather(table_pad, idx.reshape(_N_IDX))
    return conf, labels.reshape(ROWS, TOPK)


# speedup vs baseline: 1.3409x; 1.3409x over previous
import dataclasses
import functools

import jax
import jax.numpy as jnp
from jax import lax
from jax.experimental import pallas as pl
from jax.experimental.pallas import tpu as pltpu
from jax.experimental.pallas import tpu_sc as plsc

TOPK = 50
ROWS = 16384
COLS = 1000
BLOCK_ROWS = 512

_NEG_INF = float("-inf")

_SC_CORES = 2
_SC_SUBCORES = 16
_SC_WORKERS = _SC_CORES * _SC_SUBCORES
_SC_LANES = 16
_TAB_PAD = 1024

_N_IDX = ROWS * TOPK
_IDX_PER_W = _N_IDX // _SC_WORKERS


def _topk_body(xt_ref, conf_ref, idx_ref):
    x = xt_ref[...]
    iota = lax.broadcasted_iota(jnp.int32, x.shape, 0).astype(jnp.float32)
    for j in range(TOPK):
        m = jnp.max(x, axis=0)
        cand = jnp.where(x == m[None, :], iota, float(COLS + 1))
        idx = jnp.min(cand, axis=0)
        hit = cand == idx[None, :]
        conf_ref[j, :] = m
        idx_ref[j, :] = idx.astype(jnp.int32)
        x = jnp.where(hit, _NEG_INF, x)


def _tc_topk(xt):
    grid = (ROWS // BLOCK_ROWS,)
    return pl.pallas_call(
        _topk_body,
        grid=grid,
        in_specs=[pl.BlockSpec((COLS, BLOCK_ROWS), lambda i: (0, i))],
        out_specs=[
            pl.BlockSpec((TOPK, BLOCK_ROWS), lambda i: (0, i)),
            pl.BlockSpec((TOPK, BLOCK_ROWS), lambda i: (0, i)),
        ],
        out_shape=[
            jax.ShapeDtypeStruct((TOPK, ROWS), jnp.float32),
            jax.ShapeDtypeStruct((TOPK, ROWS), jnp.int32),
        ],
    )(xt)


def _sc_label_gather(table_pad, idx_flat):
    mesh = plsc.VectorSubcoreMesh(core_axis_name="c", subcore_axis_name="s")
    cp = pltpu.CompilerParams()
    if "needs_layout_passes" in pltpu.CompilerParams.__dataclass_fields__:
        cp = dataclasses.replace(cp, needs_layout_passes=False)

    @functools.partial(
        pl.kernel,
        mesh=mesh,
        compiler_params=cp,
        out_type=jax.ShapeDtypeStruct((_N_IDX,), jnp.int32),
        scratch_types=[
            pltpu.VMEM((_TAB_PAD,), jnp.int32),
            pltpu.VMEM((_IDX_PER_W,), jnp.int32),
            pltpu.VMEM((_IDX_PER_W,), jnp.int32),
        ],
    )
    def k(tab_hbm, idx_hbm, out_hbm, tab_v, idx_v, out_v):
        wid = lax.axis_index("s") * _SC_CORES + lax.axis_index("c")
        base = wid * _IDX_PER_W
        pltpu.sync_copy(tab_hbm, tab_v)
        pltpu.sync_copy(idx_hbm.at[pl.ds(base, _IDX_PER_W)], idx_v)

        @pl.loop(0, _IDX_PER_W, step=_SC_LANES)
        def _(i):
            iv = idx_v[pl.ds(i, _SC_LANES)]
            out_v[pl.ds(i, _SC_LANES)] = plsc.load_gather(tab_v, [iv])

        pltpu.sync_copy(out_v, out_hbm.at[pl.ds(base, _IDX_PER_W)])

    return k(table_pad, idx_flat)


@jax.jit
def kernel(x, label_ids):
    conf_t, idx_t = _tc_topk(x.T)
    conf = conf_t.T
    idx = idx_t.T
    table_pad = jnp.pad(label_ids, (0, _TAB_PAD - COLS))
    labels = _sc_label_gather(table_pad, idx.reshape(_N_IDX))
    return conf, labels.reshape(ROWS, TOPK)
